# Initial kernel scaffold; baseline (speedup 1.0000x reference)
import jax
import jax.numpy as jnp
from jax.experimental import pallas as pl


def kernel(idx, embedding_table):
    raise NotImplementedError("write your pallas kernel here")


# speedup vs baseline: 1.0367x; 1.0367x over previous
import functools

import jax
import jax.numpy as jnp
from jax import lax
from jax.experimental import pallas as pl
from jax.experimental.pallas import tpu as pltpu
from jax.experimental.pallas import tpu_sc as plsc

VOCAB = 1000
N_LOOKUPS = 4096 * 50
NC, NS = 2, 16
NW = NC * NS
PER_W = N_LOOKUPS // NW
CHUNK = 40
N_CHUNKS = PER_W // CHUNK


def _sc_gather(table, idx3):
  mesh = plsc.VectorSubcoreMesh(core_axis_name="c", subcore_axis_name="s",
                                num_cores=NC, num_subcores=NS)

  @functools.partial(
      pl.kernel,
      out_type=jax.ShapeDtypeStruct((N_LOOKUPS, VOCAB), jnp.float32),
      mesh=mesh,
      compiler_params=pltpu.CompilerParams(use_tc_tiling_on_sc=False),
      scratch_types=[
          pltpu.VMEM((N_CHUNKS, CHUNK), jnp.int32),
          pltpu.VMEM((CHUNK, VOCAB), jnp.float32),
          pltpu.VMEM((CHUNK, VOCAB), jnp.float32),
          pltpu.SemaphoreType.DMA,
          pltpu.SemaphoreType.DMA,
          pltpu.SemaphoreType.DMA,
          pltpu.SemaphoreType.DMA,
      ],
  )
  def k(table_hbm, idx_hbm, out_hbm, idx_v, rows0, rows1, g0, g1, w0, w1):
    wid = lax.axis_index("s") * NC + lax.axis_index("c")
    base = wid * PER_W
    pltpu.sync_copy(idx_hbm.at[wid], idx_v)
    rows = (rows0, rows1)
    gsem = (g0, g1)
    wsem = (w0, w1)

    pltpu.async_copy(table_hbm.at[idx_v.at[0]], rows0, g0)
    pltpu.async_copy(table_hbm.at[idx_v.at[1]], rows1, g1)

    def body(j, _):
      for b in range(2):
        jj = j + b
        pltpu.make_async_copy(table_hbm.at[idx_v.at[jj]], rows[b],
                              gsem[b]).wait()
        wcopy = pltpu.async_copy(
            rows[b], out_hbm.at[pl.ds(base + jj * CHUNK, CHUNK)], wsem[b])

        @pl.when(jj + 2 < N_CHUNKS)
        def _():
          wcopy.wait()
          pltpu.async_copy(table_hbm.at[idx_v.at[jj + 2]], rows[b], gsem[b])

        @pl.when(jj + 2 >= N_CHUNKS)
        def _():
          wcopy.wait()

      return ()

    lax.fori_loop(0, N_CHUNKS // 2, lambda i, c: body(i * 2, c), (),
                  unroll=False)

  return k(table, idx3)


def kernel(idx, embedding_table):
  idx3 = idx.reshape(NW, N_CHUNKS, CHUNK).astype(jnp.int32)
  out = _sc_gather(embedding_table, idx3)
  return out.reshape(idx.shape[0], idx.shape[1], VOCAB)


# speedup vs baseline: 1.1625x; 1.1213x over previous
import functools

import jax
import jax.numpy as jnp
from jax import lax
from jax.experimental import pallas as pl
from jax.experimental.pallas import tpu as pltpu
from jax.experimental.pallas import tpu_sc as plsc

VOCAB = 1000
N_LOOKUPS = 4096 * 50
NC, NS = 2, 16
NW = NC * NS
PER_W = N_LOOKUPS // NW
CHUNK = 25
N_CHUNKS = PER_W // CHUNK


def _sc_gather(table, idx3):
  mesh = plsc.VectorSubcoreMesh(core_axis_name="c", subcore_axis_name="s",
                                num_cores=NC, num_subcores=NS)

  @functools.partial(
      pl.kernel,
      out_type=jax.ShapeDtypeStruct((N_LOOKUPS, VOCAB), jnp.float32),
      mesh=mesh,
      compiler_params=pltpu.CompilerParams(use_tc_tiling_on_sc=False),
      scratch_types=[
          pltpu.VMEM((N_CHUNKS, CHUNK), jnp.int32),
          pltpu.VMEM((CHUNK, VOCAB), jnp.float32),
          pltpu.VMEM((CHUNK, VOCAB), jnp.float32),
          pltpu.VMEM_SHARED((VOCAB, VOCAB), jnp.float32),
          pltpu.SemaphoreType.DMA,
          pltpu.SemaphoreType.DMA,
          pltpu.SemaphoreType.DMA,
          pltpu.SemaphoreType.DMA,
      ],
  )
  def k(table_hbm, idx_hbm, out_hbm, idx_v, rows0, rows1, table_sp,
        g0, g1, w0, w1):
    wid = lax.axis_index("s") * NC + lax.axis_index("c")
    base = wid * PER_W

    @pl.when(lax.axis_index("s") == 0)
    def _():
      pltpu.sync_copy(table_hbm, table_sp)

    pltpu.sync_copy(idx_hbm.at[wid], idx_v)
    plsc.subcore_barrier()

    rows = (rows0, rows1)
    gsem = (g0, g1)
    wsem = (w0, w1)

    pltpu.async_copy(table_sp.at[idx_v.at[0]], rows0, g0)
    pltpu.async_copy(table_sp.at[idx_v.at[1]], rows1, g1)

    def body(j, _):
      for b in range(2):
        jj = j + b
        pltpu.make_async_copy(table_sp.at[idx_v.at[jj]], rows[b],
                              gsem[b]).wait()
        wcopy = pltpu.async_copy(
            rows[b], out_hbm.at[pl.ds(base + jj * CHUNK, CHUNK)], wsem[b])

        @pl.when(jj + 2 < N_CHUNKS)
        def _():
          wcopy.wait()
          pltpu.async_copy(table_sp.at[idx_v.at[jj + 2]], rows[b], gsem[b])

        @pl.when(jj + 2 >= N_CHUNKS)
        def _():
          wcopy.wait()

      return ()

    lax.fori_loop(0, N_CHUNKS // 2, lambda i, c: body(i * 2, c), (),
                  unroll=False)

  return k(table, idx3)


def kernel(idx, embedding_table):
  idx3 = idx.reshape(NW, N_CHUNKS, CHUNK).astype(jnp.int32)
  out = _sc_gather(embedding_table, idx3)
  return out.reshape(idx.shape[0], idx.shape[1], VOCAB)
